# Initial kernel scaffold; baseline (speedup 1.0000x reference)
import jax
import jax.numpy as jnp
from jax.experimental import pallas as pl


def kernel(atomic_numbers, e_config):
    raise NotImplementedError("write your pallas kernel here")


# speedup vs baseline: 1.8424x; 1.8424x over previous
import functools

import numpy as np
import jax
import jax.numpy as jnp
from jax import lax
from jax.experimental import pallas as pl
from jax.experimental.pallas import tpu as pltpu
from jax.experimental.pallas import tpu_sc as plsc

N_AT = 100000
NE = 119
D = 24
L = 16

NC = 2
NS = 16
NW = NC * NS

ATOMS_PER_W = N_AT // NW
WORDS_PER_W = ATOMS_PER_W * D
M_ITERS = -(-WORDS_PER_W // 48)
OUT_V_WORDS = M_ITERS * 48
TAB_WORDS = NE * D

IDX_WIN = 3136
IDX_ALLOC = 3200
MAX_ASTART = N_AT - IDX_WIN

@functools.partial(
    pl.kernel,
    out_type=jax.ShapeDtypeStruct((N_AT * D,), jnp.float32),
    mesh=plsc.VectorSubcoreMesh(core_axis_name="c", subcore_axis_name="s"),
    scratch_types=[
        pltpu.VMEM((TAB_WORDS,), jnp.float32),
        pltpu.VMEM((IDX_ALLOC,), jnp.int32),
        pltpu.VMEM((OUT_V_WORDS,), jnp.float32),
    ],
    compiler_params=pltpu.CompilerParams(needs_layout_passes=False),
)
def _gather_kernel(tab_hbm, idx_hbm, out_hbm, tab_v, idx_v, out_v):
    cid = lax.axis_index("c")
    sid = lax.axis_index("s")
    wid = sid * NC + cid

    start = wid * ATOMS_PER_W
    astart = jnp.minimum(start & ~7, MAX_ASTART)
    astart = pl.multiple_of(astart, 8)
    loff = start - astart

    pltpu.sync_copy(tab_hbm, tab_v)
    pltpu.sync_copy(idx_hbm.at[pl.ds(astart, IDX_WIN)], idx_v.at[pl.ds(0, IDX_WIN)])

    iota = lax.iota(jnp.int32, L)
    dqs, rrs = [], []
    for p in (0, 16, 32):
        t = p + iota
        dq = jnp.where(t >= D, 1, 0)
        dqs.append(dq)
        rrs.append(t - dq * D)

    def step(m, carry):
        obase = m * 48
        qoff = 2 * m + loff
        for i, p in enumerate((0, 16, 32)):
            q = dqs[i] + qoff
            z = plsc.load_gather(idx_v, [q])
            z = jnp.minimum(jnp.maximum(z, 0), NE - 1)
            src = z * D + rrs[i]
            v = plsc.load_gather(tab_v, [src])
            out_v[pl.ds(obase + p, L)] = v
        return carry

    lax.fori_loop(0, M_ITERS, step, 0)

    pltpu.sync_copy(
        out_v.at[pl.ds(0, WORDS_PER_W)],
        out_hbm.at[pl.ds(start * D, WORDS_PER_W)],
    )


@jax.jit
def kernel(atomic_numbers, e_config):
    out_flat = _gather_kernel(e_config.reshape(-1), atomic_numbers)
    return out_flat.reshape(N_AT, D)


# speedup vs baseline: 2.9932x; 1.6247x over previous
import functools

import numpy as np
import jax
import jax.numpy as jnp
from jax import lax
from jax.experimental import pallas as pl
from jax.experimental.pallas import tpu as pltpu
from jax.experimental.pallas import tpu_sc as plsc

N_AT = 100000
NE = 119
D = 24
L = 16

NC = 2
NS = 16
NW = NC * NS

ATOMS_PER_W = N_AT // NW
WORDS_PER_W = ATOMS_PER_W * D
M_ITERS = -(-WORDS_PER_W // 48)
OUT_V_WORDS = M_ITERS * 48
TAB_WORDS = NE * D

IDX_WIN = 3136
IDX_ALLOC = 3200
MAX_ASTART = N_AT - IDX_WIN

@functools.partial(
    pl.kernel,
    out_type=jax.ShapeDtypeStruct((N_AT * D,), jnp.float32),
    mesh=plsc.VectorSubcoreMesh(core_axis_name="c", subcore_axis_name="s"),
    scratch_types=[
        pltpu.VMEM((TAB_WORDS,), jnp.float32),
        pltpu.VMEM((IDX_ALLOC,), jnp.int32),
        pltpu.VMEM((OUT_V_WORDS,), jnp.float32),
    ],
    compiler_params=pltpu.CompilerParams(needs_layout_passes=False),
)
def _gather_kernel(tab_hbm, idx_hbm, out_hbm, tab_v, idx_v, out_v):
    cid = lax.axis_index("c")
    sid = lax.axis_index("s")
    wid = sid * NC + cid

    start = wid * ATOMS_PER_W
    astart = jnp.minimum(start & ~7, MAX_ASTART)
    astart = pl.multiple_of(astart, 8)
    loff = start - astart

    pltpu.sync_copy(tab_hbm, tab_v)
    pltpu.sync_copy(idx_hbm.at[pl.ds(astart, IDX_WIN)], idx_v.at[pl.ds(0, IDX_WIN)])

    iota = lax.iota(jnp.int32, L)
    dqs, rrs = [], []
    for p in (0, 16, 32):
        t = p + iota
        dq = jnp.where(t >= D, 1, 0)
        dqs.append(dq)
        rrs.append(t - dq * D)

    @plsc.parallel_loop(0, M_ITERS, 1, unroll=8)
    def _loop(m):
        obase = m * 48
        qoff = 2 * m + loff
        for i, p in enumerate((0, 16, 32)):
            q = dqs[i] + qoff
            z = plsc.load_gather(idx_v, [q])
            z = jnp.minimum(jnp.maximum(z, 0), NE - 1)
            src = z * D + rrs[i]
            v = plsc.load_gather(tab_v, [src])
            out_v[pl.ds(obase + p, L)] = v

    pltpu.sync_copy(
        out_v.at[pl.ds(0, WORDS_PER_W)],
        out_hbm.at[pl.ds(start * D, WORDS_PER_W)],
    )


@jax.jit
def kernel(atomic_numbers, e_config):
    out_flat = _gather_kernel(e_config.reshape(-1), atomic_numbers)
    return out_flat.reshape(N_AT, D)


# speedup vs baseline: 8.1041x; 2.7075x over previous
import functools

import numpy as np
import jax
import jax.numpy as jnp
from jax import lax
from jax.experimental import pallas as pl
from jax.experimental.pallas import tpu as pltpu
from jax.experimental.pallas import tpu_sc as plsc

N_AT = 100000
NE = 119
D = 24
L = 16

NC = 2
NS = 16
NW = NC * NS

N_PAD = 100096
APW = 3200
APW_LAST = N_PAD - (NW - 1) * APW
N_CHUNK = APW // L
MAX_ASTART = N_AT - APW
IDX_ALLOC = APW + 2432
TAB_WORDS = NE * D


@functools.partial(
    pl.kernel,
    out_type=jax.ShapeDtypeStruct((D, N_PAD), jnp.float32),
    mesh=plsc.VectorSubcoreMesh(core_axis_name="c", subcore_axis_name="s"),
    scratch_types=[
        pltpu.VMEM((TAB_WORDS,), jnp.float32),
        pltpu.VMEM((IDX_ALLOC,), jnp.int32),
        pltpu.VMEM((D, APW), jnp.float32),
    ],
    compiler_params=pltpu.CompilerParams(needs_layout_passes=False),
)
def _gather_kernel(tab_hbm, idx_hbm, out_hbm, tab_v, idx_v, out_v):
    cid = lax.axis_index("c")
    sid = lax.axis_index("s")
    wid = sid * NC + cid

    i0 = wid * APW
    astart = jnp.minimum(i0, MAX_ASTART)
    astart = pl.multiple_of(astart, 8)
    loff = i0 - astart

    pltpu.sync_copy(tab_hbm, tab_v)
    pltpu.sync_copy(idx_hbm.at[pl.ds(astart, APW)], idx_v.at[pl.ds(0, APW)])

    @plsc.parallel_loop(0, N_CHUNK, 1, unroll=2)
    def _loop(c):
        zvec = idx_v[pl.ds(loff + c * L, L)]
        z24 = jnp.minimum(jnp.maximum(zvec, 0), NE - 1) * D
        for j in range(D):
            v = plsc.load_gather(tab_v, [z24 + j])
            out_v[j, pl.ds(c * L, L)] = v

    @pl.when(wid < NW - 1)
    def _():
        pltpu.sync_copy(out_v, out_hbm.at[:, pl.ds(i0, APW)])

    @pl.when(wid == NW - 1)
    def _():
        pltpu.sync_copy(
            out_v.at[:, pl.ds(0, APW_LAST)],
            out_hbm.at[:, pl.ds(i0, APW_LAST)],
        )


@jax.jit
def kernel(atomic_numbers, e_config):
    return _gather_kernel(e_config.reshape(-1), atomic_numbers).T[:N_AT]
